# Initial kernel scaffold; baseline (speedup 1.0000x reference)
import jax
import jax.numpy as jnp
from jax.experimental import pallas as pl


def kernel(x, edge_index, batch, Wc1, bc1, Wc2, bc2, Wc3, bc3, Wc4, bc4, Wf1, bf1, Wf2, bf2, Wf3, bf3, Wo, bo):
    raise NotImplementedError("write your pallas kernel here")


# speedup vs baseline: 8.6455x; 8.6455x over previous
import functools

import jax
import jax.numpy as jnp
from jax import lax
from jax.experimental import pallas as pl
from jax.experimental.pallas import tpu as pltpu
from jax.experimental.pallas import tpu_sc as plsc

NC = 2
NS = 16
NW = NC * NS


def _sub_rows(n):
    rps = ((n + NS - 1) // NS + 7) // 8 * 8
    last = n - (NS - 1) * rps
    assert last > 0 and last % 8 == 0 and rps % 8 == 0
    return rps, last


@functools.lru_cache(maxsize=None)
def _make_prop(n, C, E, K):
    ew = E // NW
    nblk = ew // K
    rps, last = _sub_rows(n)

    mesh = plsc.VectorSubcoreMesh(
        core_axis_name="c", subcore_axis_name="s",
        num_cores=NC, num_subcores=NS)

    def body(g_hbm, src_hbm, dst_hbm, zero_hbm, out_hbm,
             sidx_v, didx_v, rows_v, acc_sh, sem):
        cid = lax.axis_index("c")
        sid = lax.axis_index("s")
        wid = sid * NC + cid
        r0 = sid * rps

        @pl.when(sid < NS - 1)
        def _():
            pltpu.sync_copy(zero_hbm, acc_sh.at[pl.ds(r0, rps)])

        @pl.when(sid == NS - 1)
        def _():
            pltpu.sync_copy(zero_hbm.at[pl.ds(0, last)],
                            acc_sh.at[pl.ds((NS - 1) * rps, last)])

        plsc.subcore_barrier()

        base = wid * ew

        def step(i, carry):
            off = base + i * K
            pltpu.sync_copy(src_hbm.at[pl.ds(off, K)], sidx_v)
            pltpu.sync_copy(dst_hbm.at[pl.ds(off, K)], didx_v)
            pltpu.async_copy(g_hbm.at[sidx_v], rows_v, sem).wait()
            pltpu.sync_copy(rows_v, acc_sh.at[didx_v], add=True)
            return carry

        lax.fori_loop(0, nblk, step, 0)
        plsc.subcore_barrier()

        @pl.when(sid < NS - 1)
        def _():
            pltpu.sync_copy(acc_sh.at[pl.ds(r0, rps)],
                            out_hbm.at[cid, pl.ds(r0, rps)])

        @pl.when(sid == NS - 1)
        def _():
            pltpu.sync_copy(acc_sh.at[pl.ds((NS - 1) * rps, last)],
                            out_hbm.at[cid, pl.ds((NS - 1) * rps, last)])

    return pl.kernel(
        body,
        out_type=jax.ShapeDtypeStruct((NC, n, C), jnp.float32),
        mesh=mesh,
        compiler_params=pltpu.CompilerParams(use_tc_tiling_on_sc=False),
        scratch_types=[
            pltpu.VMEM((K,), jnp.int32),
            pltpu.VMEM((K,), jnp.int32),
            pltpu.VMEM((K, C), jnp.float32),
            pltpu.VMEM_SHARED((n, C), jnp.float32),
            pltpu.SemaphoreType.DMA,
        ],
    )


def _prop(g, src, dst, zero32):
    n, d = g.shape
    outs = []
    for j in range(d // 32):
        chunk = lax.slice(g, (0, 32 * j), (n, 32 * j + 32))
        outs.append(_make_prop(n, 32, src.shape[0], 200)(
            chunk, src, dst, zero32))
    return jnp.concatenate(outs, axis=-1)


def _k1_body(deg_ref, x_ref, dis_ref, g1_ref):
    deg = deg_ref[0, :, 0:1] + deg_ref[1, :, 0:1] + 1.0
    dis = lax.rsqrt(deg)
    dis_ref[...] = dis
    g1_ref[...] = x_ref[...] * dis


def _k_l1_body(t_ref, g_ref, dis_ref, w1_ref, b1_ref, w2_ref, out_ref):
    dis = dis_ref[...]
    q = (t_ref[0] + t_ref[1] + g_ref[...]) * dis
    h = jnp.maximum(jnp.dot(q, w1_ref[...],
                            preferred_element_type=jnp.float32)
                    + b1_ref[...], 0.0)
    out_ref[...] = dis * jnp.dot(h, w2_ref[...],
                                 preferred_element_type=jnp.float32)


def _k_mid_body(t_ref, g_ref, dis_ref, b_ref, w_ref, out_ref):
    dis = dis_ref[...]
    h = jnp.maximum((t_ref[0] + t_ref[1] + g_ref[...]) * dis + b_ref[...],
                    0.0)
    out_ref[...] = dis * jnp.dot(h, w_ref[...],
                                 preferred_element_type=jnp.float32)


def _k_pool_body(t_ref, g_ref, dis_ref, b_ref, batch_ref, pool_ref):
    i = pl.program_id(0)

    @pl.when(i == 0)
    def _():
        pool_ref[...] = jnp.full_like(pool_ref, -jnp.inf)

    dis = dis_ref[...]
    h = jnp.maximum((t_ref[0] + t_ref[1] + g_ref[...]) * dis + b_ref[...],
                    0.0)
    nb = pool_ref.shape[0]
    batchv = batch_ref[...]
    rows = [jnp.max(jnp.where(batchv == g, h, -jnp.inf), axis=0,
                    keepdims=True)
            for g in range(nb)]
    pool_ref[...] = jnp.maximum(pool_ref[...],
                                jnp.concatenate(rows, axis=0))


def _k_head_body(p_ref, w1_ref, b1_ref, w2_ref, b2_ref, w3_ref, b3_ref,
                 wo_ref, bo_ref, out_ref, feat_ref):
    p = p_ref[...]
    p = jnp.where(jnp.isfinite(p), p, 0.0)
    g = jnp.maximum(jnp.dot(p, w1_ref[...],
                            preferred_element_type=jnp.float32)
                    + b1_ref[...], 0.0)
    g = jnp.maximum(jnp.dot(g, w2_ref[...],
                            preferred_element_type=jnp.float32)
                    + b2_ref[...], 0.0)
    feat = jnp.maximum(jnp.dot(g, w3_ref[...],
                               preferred_element_type=jnp.float32)
                       + b3_ref[...], 0.0)
    feat_ref[...] = feat
    out_ref[...] = jnp.dot(feat, wo_ref[...],
                           preferred_element_type=jnp.float32) + bo_ref[...]


def kernel(x, edge_index, batch, Wc1, bc1, Wc2, bc2, Wc3, bc3, Wc4, bc4,
           Wf1, bf1, Wf2, bf2, Wf3, bf3, Wo, bo):
    n = x.shape[0]
    E = edge_index.shape[1]
    src = edge_index[0]
    dst = edge_index[1]

    rps0, _ = _sub_rows(n)
    zero32 = jnp.zeros((rps0, 32), jnp.float32)

    ones_tab = jnp.ones((n, 32), jnp.float32)
    degp = _make_prop(n, 32, E, 200)(ones_tab, src, dst, zero32)

    x_pad = jnp.pad(x, ((0, 0), (0, 96 - x.shape[1])))
    W1p = jnp.pad(Wc1, ((0, 96 - Wc1.shape[0]), (0, 0)))

    RB = 2000
    grid = (n // RB,)
    seq = pltpu.CompilerParams(dimension_semantics=("arbitrary",))

    dis, g1 = pl.pallas_call(
        _k1_body,
        grid=grid,
        in_specs=[
            pl.BlockSpec((NC, RB, 32), lambda i: (0, i, 0)),
            pl.BlockSpec((RB, 96), lambda i: (i, 0)),
        ],
        out_specs=[
            pl.BlockSpec((RB, 1), lambda i: (i, 0)),
            pl.BlockSpec((RB, 96), lambda i: (i, 0)),
        ],
        out_shape=[
            jax.ShapeDtypeStruct((n, 1), jnp.float32),
            jax.ShapeDtypeStruct((n, 96), jnp.float32),
        ],
        compiler_params=seq,
    )(degp, x_pad)

    t1 = _prop(g1, src, dst, zero32)
    g2 = pl.pallas_call(
        _k_l1_body,
        grid=grid,
        in_specs=[
            pl.BlockSpec((NC, RB, 96), lambda i: (0, i, 0)),
            pl.BlockSpec((RB, 96), lambda i: (i, 0)),
            pl.BlockSpec((RB, 1), lambda i: (i, 0)),
            pl.BlockSpec((96, 256), lambda i: (0, 0)),
            pl.BlockSpec((1, 256), lambda i: (0, 0)),
            pl.BlockSpec((256, 128), lambda i: (0, 0)),
        ],
        out_specs=pl.BlockSpec((RB, 128), lambda i: (i, 0)),
        out_shape=jax.ShapeDtypeStruct((n, 128), jnp.float32),
        compiler_params=seq,
    )(t1, g1, dis, W1p, bc1.reshape(1, -1), Wc2)

    def mid(g, b, Wnext):
        t = _prop(g, src, dst, zero32)
        cin = g.shape[1]
        cout = Wnext.shape[1]
        return pl.pallas_call(
            _k_mid_body,
            grid=grid,
            in_specs=[
                pl.BlockSpec((NC, RB, cin), lambda i: (0, i, 0)),
                pl.BlockSpec((RB, cin), lambda i: (i, 0)),
                pl.BlockSpec((RB, 1), lambda i: (i, 0)),
                pl.BlockSpec((1, cin), lambda i: (0, 0)),
                pl.BlockSpec((cin, cout), lambda i: (0, 0)),
            ],
            out_specs=pl.BlockSpec((RB, cout), lambda i: (i, 0)),
            out_shape=jax.ShapeDtypeStruct((n, cout), jnp.float32),
            compiler_params=seq,
        )(t, g, dis, b.reshape(1, -1), Wnext)

    g3 = mid(g2, bc2, Wc3)
    g4 = mid(g3, bc3, Wc4)

    t4 = _prop(g4, src, dst, zero32)
    PB = 2000
    pooled = pl.pallas_call(
        _k_pool_body,
        grid=(n // PB,),
        in_specs=[
            pl.BlockSpec((NC, PB, 32), lambda i: (0, i, 0)),
            pl.BlockSpec((PB, 32), lambda i: (i, 0)),
            pl.BlockSpec((PB, 1), lambda i: (i, 0)),
            pl.BlockSpec((1, 32), lambda i: (0, 0)),
            pl.BlockSpec((PB, 1), lambda i: (i, 0)),
        ],
        out_specs=pl.BlockSpec((64, 32), lambda i: (0, 0)),
        out_shape=jax.ShapeDtypeStruct((64, 32), jnp.float32),
        compiler_params=seq,
    )(t4, g4, dis, bc4.reshape(1, -1), batch.reshape(n, 1))

    out, feat = pl.pallas_call(
        _k_head_body,
        in_specs=[pl.BlockSpec(s, lambda: (0,) * len(s)) for s in [
            (64, 32), (32, 1024), (1, 1024), (1024, 512), (1, 512),
            (512, 256), (1, 256), (256, 1), (1, 1)]],
        out_specs=[
            pl.BlockSpec((64, 1), lambda: (0, 0)),
            pl.BlockSpec((64, 256), lambda: (0, 0)),
        ],
        out_shape=[
            jax.ShapeDtypeStruct((64, 1), jnp.float32),
            jax.ShapeDtypeStruct((64, 256), jnp.float32),
        ],
    )(pooled, Wf1, bf1.reshape(1, -1), Wf2, bf2.reshape(1, -1),
      Wf3, bf3.reshape(1, -1), Wo, bo.reshape(1, -1))

    return (out, feat)
